# baseline (speedup 1.0000x reference)
import functools

import jax
import jax.numpy as jnp
from jax import lax
from jax.experimental import pallas as pl
from jax.experimental.pallas import tpu as pltpu
from jax.experimental.pallas import tpu_sc as plsc


def _positional_encoding(seq_length, embedding_dim):
    position = jnp.arange(seq_length, dtype=jnp.float32)[:, None]
    div = 10000.0 ** (
        jnp.arange(0, embedding_dim, 2, dtype=jnp.float32) / embedding_dim
    )
    pe = jnp.zeros((seq_length, embedding_dim), dtype=jnp.float32)
    pe = pe.at[:, 0::2].set(jnp.sin(position / div))
    pe = pe.at[:, 1::2].set(jnp.cos(position / div))
    return pe


def _num_subcores():
    try:
        info = plsc.get_sparse_core_info()
        return info.num_cores, info.num_subcores
    except Exception:
        return 2, 16


@functools.partial(jax.jit, static_argnames=())
def kernel(x, table):
    B, S = x.shape
    V, D = table.shape
    NC, NS = _num_subcores()
    NW = NC * NS
    assert B % NW == 0, (B, NW)
    rows_per_w = B // NW

    CK = S
    while CK > 128:
        for cand in range(min(128, CK - 1), 0, -1):
            if S % cand == 0:
                CK = cand
                break
    NCK = S // CK

    lanes = 16
    assert D % lanes == 0

    pe = _positional_encoding(S, D)
    x3 = x.astype(jnp.int32).reshape(B, NCK, CK)

    mesh = plsc.VectorSubcoreMesh(
        core_axis_name="c", subcore_axis_name="s", num_cores=NC, num_subcores=NS
    )

    @functools.partial(
        pl.kernel,
        out_type=jax.ShapeDtypeStruct((B, S, D), jnp.float32),
        mesh=mesh,
        compiler_params=pltpu.CompilerParams(use_tc_tiling_on_sc=False),
        scratch_types=[
            pltpu.VMEM((NCK, CK), jnp.int32),
            pltpu.VMEM((S, D), jnp.float32),
            pltpu.VMEM((S, D), jnp.float32),
            pltpu.SemaphoreType.DMA,
        ],
    )
    def _emb(x_hbm, table_hbm, pe_hbm, out_hbm, idx_v, rows_v, pe_v, sem):
        wid = lax.axis_index("s") * NC + lax.axis_index("c")
        base = wid * rows_per_w
        pltpu.sync_copy(pe_hbm, pe_v)

        @pl.loop(0, rows_per_w)
        def _row(r):
            row = base + r
            pltpu.sync_copy(x_hbm.at[row], idx_v)
            for h in range(NCK):
                pltpu.async_copy(
                    table_hbm.at[idx_v.at[h]],
                    rows_v.at[pl.ds(h * CK, CK)],
                    sem,
                ).wait()

            @pl.loop(0, S)
            def _s(s):
                for d in range(D // lanes):
                    v = pe_v[s, pl.ds(d * lanes, lanes)]
                    plsc.addupdate(rows_v.at[s, pl.ds(d * lanes, lanes)], v)

            pltpu.sync_copy(rows_v, out_hbm.at[row])

    return _emb(x3, table, pe)


# speedup vs baseline: 1.2241x; 1.2241x over previous
import functools

import jax
import jax.numpy as jnp
from jax import lax
from jax.experimental import pallas as pl
from jax.experimental.pallas import tpu as pltpu
from jax.experimental.pallas import tpu_sc as plsc


def _positional_encoding(seq_length, embedding_dim):
    position = jnp.arange(seq_length, dtype=jnp.float32)[:, None]
    div = 10000.0 ** (
        jnp.arange(0, embedding_dim, 2, dtype=jnp.float32) / embedding_dim
    )
    pe = jnp.zeros((seq_length, embedding_dim), dtype=jnp.float32)
    pe = pe.at[:, 0::2].set(jnp.sin(position / div))
    pe = pe.at[:, 1::2].set(jnp.cos(position / div))
    return pe


def _num_subcores():
    try:
        info = plsc.get_sparse_core_info()
        return info.num_cores, info.num_subcores
    except Exception:
        return 2, 16


def kernel(x, table):
    B, S = x.shape
    V, D = table.shape
    NC, NS = _num_subcores()
    NW = NC * NS
    assert B % NW == 0, (B, NW)
    rows_per_w = B // NW

    CK = S
    if CK > 128:
        for cand in range(128, 0, -1):
            if S % cand == 0:
                CK = cand
                break
    NCK = S // CK

    lanes = 16
    assert D % lanes == 0

    P = 4
    assert rows_per_w % P == 0 and rows_per_w >= 2 * P

    pe = _positional_encoding(S, D)
    x2 = x.astype(jnp.int32).reshape(B * NCK, CK)

    mesh = plsc.VectorSubcoreMesh(
        core_axis_name="c", subcore_axis_name="s", num_cores=NC, num_subcores=NS
    )

    @functools.partial(
        pl.kernel,
        out_type=jax.ShapeDtypeStruct((B, S, D), jnp.float32),
        mesh=mesh,
        compiler_params=pltpu.CompilerParams(use_tc_tiling_on_sc=False),
        scratch_types=[
            pltpu.VMEM((rows_per_w * NCK, CK), jnp.int32),
            pltpu.VMEM((S, D), jnp.float32),
        ]
        + [pltpu.VMEM((S, D), jnp.float32) for _ in range(P)]
        + [pltpu.SemaphoreType.DMA for _ in range(2 * P)],
    )
    def _emb(x_hbm, table_hbm, pe_hbm, out_hbm, idx_v, pe_v, *rest):
        rows = rest[:P]
        gsem = rest[P : 2 * P]
        osem = rest[2 * P : 3 * P]

        wid = lax.axis_index("s") * NC + lax.axis_index("c")
        base = wid * rows_per_w
        pltpu.sync_copy(x_hbm.at[pl.ds(base * NCK, rows_per_w * NCK)], idx_v)
        pltpu.sync_copy(pe_hbm, pe_v)

        def fire_gather(r, k):
            for h in range(NCK):
                pltpu.async_copy(
                    table_hbm.at[idx_v.at[r * NCK + h]],
                    rows[k].at[pl.ds(h * CK, CK)],
                    gsem[k],
                )

        def wait_gather(k):
            for _ in range(NCK):
                pltpu.make_async_copy(
                    out_hbm.at[0, pl.ds(0, CK)],
                    rows[k].at[pl.ds(0, CK)],
                    gsem[k],
                ).wait()

        def wait_write(k):
            pltpu.make_async_copy(rows[k], out_hbm.at[0], osem[k]).wait()

        for k in range(P):
            fire_gather(k, k)

        @pl.loop(0, rows_per_w // P)
        def _blk(gb):
            for k in range(P):
                g = gb * P + k
                kprev = (k - 1) % P

                @pl.when(jnp.logical_and(g >= 1, g <= rows_per_w - P))
                def _():
                    wait_write(kprev)
                    fire_gather(g + P - 1, kprev)

                wait_gather(k)

                @pl.loop(0, S)
                def _s(s):
                    for d in range(D // lanes):
                        v = pe_v[s, pl.ds(d * lanes, lanes)]
                        plsc.addupdate(rows[k].at[s, pl.ds(d * lanes, lanes)], v)

                pltpu.async_copy(rows[k], out_hbm.at[base + g], osem[k])

        for k in range(P):
            wait_write(k)

    return _emb(x2, table, pe)
